# baseline (speedup 1.0000x reference)
import functools

import jax
import jax.numpy as jnp
from jax import lax
from jax.experimental import pallas as pl
from jax.experimental.pallas import tpu as pltpu
from jax.experimental.pallas import tpu_sc as plsc

N_COLORS = 1000
EMB_DIM = 50
VOCAB = 100000
BATCH = 1024

V_TILE = 2048
NV = (VOCAB + V_TILE - 1) // V_TILE
B_HALF = BATCH // 2

NEG_BIG = -1e30


_SC_TILES = 32
_B_PER_TILE = BATCH // _SC_TILES
_SC_D = 128


@functools.cache
def _make_sc_gather():
    mesh = plsc.VectorSubcoreMesh(core_axis_name="c", subcore_axis_name="s")

    @functools.partial(
        pl.kernel,
        mesh=mesh,
        out_type=jax.ShapeDtypeStruct((BATCH, _SC_D), jnp.float32),
        scratch_types=[
            pltpu.VMEM((_B_PER_TILE,), jnp.int32),
            pltpu.VMEM((_B_PER_TILE, _SC_D), jnp.float32),
            pltpu.SemaphoreType.DMA,
        ],
    )
    def _sc_gather(table_hbm, idx_hbm, out_hbm, idx_v, rows_v, sem):
        wid = lax.axis_index("s") * 2 + lax.axis_index("c")
        base = wid * _B_PER_TILE
        pltpu.sync_copy(idx_hbm.at[pl.ds(base, _B_PER_TILE)], idx_v)
        pltpu.async_copy(table_hbm.at[idx_v], rows_v, sem).wait()
        pltpu.sync_copy(rows_v, out_hbm.at[pl.ds(base, _B_PER_TILE)])

    return _sc_gather


def _leaky_logits(e_ref, w_ref, b_ref):
    e = e_ref[:, :EMB_DIM]
    e = jnp.where(e >= 0, e, 0.01 * e)
    return lax.dot_general(
        e, w_ref[...], (((1,), (1,)), ((), ())),
        preferred_element_type=jnp.float32,
    ) + b_ref[...]


def _lse_body(e_ref, w_ref, b_ref, lse_ref, m_ref, s_ref):
    j = pl.program_id(1)

    @pl.when(j == 0)
    def _():
        m_ref[...] = jnp.full_like(m_ref, -jnp.inf)
        s_ref[...] = jnp.zeros_like(s_ref)

    logits = _leaky_logits(e_ref, w_ref, b_ref)
    col = j * V_TILE + lax.broadcasted_iota(jnp.int32, logits.shape, 1)
    logits = jnp.where(col < VOCAB, logits, NEG_BIG)
    m_old = m_ref[...]
    m_new = jnp.maximum(m_old, jnp.max(logits, axis=1, keepdims=True))
    s_ref[...] = s_ref[...] * jnp.exp(m_old - m_new) + jnp.sum(
        jnp.exp(logits - m_new), axis=1, keepdims=True)
    m_ref[...] = m_new

    @pl.when(j == pl.num_programs(1) - 1)
    def _():
        lse_ref[...] = m_ref[...] + jnp.log(s_ref[...])


def _lse_pass(e, W, b2):
    return pl.pallas_call(
        _lse_body,
        grid=(2, NV),
        in_specs=[
            pl.BlockSpec((B_HALF, _SC_D), lambda i, j: (i, 0)),
            pl.BlockSpec((V_TILE, EMB_DIM), lambda i, j: (j, 0)),
            pl.BlockSpec((1, V_TILE), lambda i, j: (0, j)),
        ],
        out_specs=pl.BlockSpec((B_HALF, 1), lambda i, j: (i, 0)),
        out_shape=jax.ShapeDtypeStruct((BATCH, 1), jnp.float32),
        scratch_shapes=[
            pltpu.VMEM((B_HALF, 1), jnp.float32),
            pltpu.VMEM((B_HALF, 1), jnp.float32),
        ],
        compiler_params=pltpu.CompilerParams(
            dimension_semantics=("parallel", "arbitrary")),
    )(e, W, b2)


def _out_body(e_ref, w_ref, b_ref, lse_ref, o_ref):
    res = _leaky_logits(e_ref, w_ref, b_ref) - lse_ref[...]
    o_ref[...] = res.reshape(B_HALF, 1, V_TILE)


def _out_pass(e, W, b2, lse):
    return pl.pallas_call(
        _out_body,
        grid=(2, NV),
        in_specs=[
            pl.BlockSpec((B_HALF, _SC_D), lambda i, j: (i, 0)),
            pl.BlockSpec((V_TILE, EMB_DIM), lambda i, j: (j, 0)),
            pl.BlockSpec((1, V_TILE), lambda i, j: (0, j)),
            pl.BlockSpec((B_HALF, 1), lambda i, j: (i, 0)),
        ],
        out_specs=pl.BlockSpec((B_HALF, 1, V_TILE), lambda i, j: (i, 0, j)),
        out_shape=jax.ShapeDtypeStruct((BATCH, 1, VOCAB), jnp.float32),
        compiler_params=pltpu.CompilerParams(
            dimension_semantics=("parallel", "arbitrary")),
    )(e, W, b2, lse)


def kernel(x, emb_table, W, b):
    x0 = x[:, 0].astype(jnp.int32)
    table_pad = jnp.pad(emb_table, ((0, 0), (0, _SC_D - EMB_DIM)))
    b2 = b.reshape(1, VOCAB)

    e = _make_sc_gather()(table_pad, x0)
    lse = _lse_pass(e, W, b2)
    return _out_pass(e, W, b2, lse)


# speedup vs baseline: 3.5135x; 3.5135x over previous
import functools

import jax
import jax.numpy as jnp
from jax import lax
from jax.experimental import pallas as pl
from jax.experimental.pallas import tpu as pltpu
from jax.experimental.pallas import tpu_sc as plsc

N_COLORS = 1000
EMB_DIM = 50
VOCAB = 100000
BATCH = 1024

V_TILE = 2048
NV = (VOCAB + V_TILE - 1) // V_TILE
B_HALF = BATCH // 2

NEG_BIG = -1e30


_SC_TILES = 32
_B_PER_TILE = BATCH // _SC_TILES
_SC_D = 128


@functools.cache
def _make_sc_gather():
    mesh = plsc.VectorSubcoreMesh(core_axis_name="c", subcore_axis_name="s")

    @functools.partial(
        pl.kernel,
        mesh=mesh,
        out_type=jax.ShapeDtypeStruct((BATCH, _SC_D), jnp.float32),
        scratch_types=[
            pltpu.VMEM((_B_PER_TILE,), jnp.int32),
            pltpu.VMEM((_B_PER_TILE, _SC_D), jnp.float32),
            pltpu.SemaphoreType.DMA,
        ],
    )
    def _sc_gather(table_hbm, idx_hbm, out_hbm, idx_v, rows_v, sem):
        wid = lax.axis_index("s") * 2 + lax.axis_index("c")
        base = wid * _B_PER_TILE
        pltpu.sync_copy(idx_hbm.at[pl.ds(base, _B_PER_TILE)], idx_v)
        pltpu.async_copy(table_hbm.at[idx_v], rows_v, sem).wait()
        pltpu.sync_copy(rows_v, out_hbm.at[pl.ds(base, _B_PER_TILE)])

    return _sc_gather


def _leaky_logits_t(e_ref, wt_ref, b_ref):
    e = e_ref[:, :EMB_DIM]
    e = jnp.where(e >= 0, e, 0.01 * e)
    lt = lax.dot_general(
        wt_ref[...], e, (((0,), (1,)), ((), ())),
        preferred_element_type=jnp.float32,
    )
    return lt + b_ref[...].reshape(V_TILE, 1)


def _lse_body(e_ref, w_ref, b_ref, lse_ref, m_ref, s_ref):
    j = pl.program_id(1)

    @pl.when(j == 0)
    def _():
        m_ref[...] = jnp.full_like(m_ref, -jnp.inf)
        s_ref[...] = jnp.zeros_like(s_ref)

    logits = _leaky_logits_t(e_ref, w_ref, b_ref)
    row = j * V_TILE + lax.broadcasted_iota(jnp.int32, logits.shape, 0)
    logits = jnp.where(row < VOCAB, logits, NEG_BIG)
    m_old = m_ref[...]
    m_new = jnp.maximum(m_old, jnp.max(logits, axis=0, keepdims=True))
    s_ref[...] = s_ref[...] * jnp.exp(m_old - m_new) + jnp.sum(
        jnp.exp(logits - m_new), axis=0, keepdims=True)
    m_ref[...] = m_new

    @pl.when(j == pl.num_programs(1) - 1)
    def _():
        lse_ref[...] = m_ref[...] + jnp.log(s_ref[...])


def _lse_pass(e, W, b2):
    return pl.pallas_call(
        _lse_body,
        grid=(2, NV),
        in_specs=[
            pl.BlockSpec((B_HALF, _SC_D), lambda i, j: (i, 0)),
            pl.BlockSpec((EMB_DIM, V_TILE), lambda i, j: (0, j)),
            pl.BlockSpec((1, V_TILE), lambda i, j: (0, j)),
        ],
        out_specs=pl.BlockSpec((1, B_HALF), lambda i, j: (0, i)),
        out_shape=jax.ShapeDtypeStruct((1, BATCH), jnp.float32),
        scratch_shapes=[
            pltpu.VMEM((1, B_HALF), jnp.float32),
            pltpu.VMEM((1, B_HALF), jnp.float32),
        ],
        compiler_params=pltpu.CompilerParams(
            dimension_semantics=("parallel", "arbitrary")),
    )(e, W, b2)


def _out_body(e_ref, w_ref, b_ref, lse_ref, o_ref):
    o_ref[...] = _leaky_logits_t(e_ref, w_ref, b_ref) - lse_ref[...]


def _out_pass(e, W, b2, lse):
    return pl.pallas_call(
        _out_body,
        grid=(2, NV),
        in_specs=[
            pl.BlockSpec((B_HALF, _SC_D), lambda i, j: (i, 0)),
            pl.BlockSpec((EMB_DIM, V_TILE), lambda i, j: (0, j)),
            pl.BlockSpec((1, V_TILE), lambda i, j: (0, j)),
            pl.BlockSpec((1, B_HALF), lambda i, j: (0, i)),
        ],
        out_specs=pl.BlockSpec((V_TILE, B_HALF), lambda i, j: (j, i)),
        out_shape=jax.ShapeDtypeStruct((VOCAB, BATCH), jnp.float32),
        compiler_params=pltpu.CompilerParams(
            dimension_semantics=("parallel", "arbitrary")),
    )(e, W, b2, lse)


def kernel(x, emb_table, W, b):
    x0 = x[:, 0].astype(jnp.int32)
    table_pad = jnp.pad(emb_table, ((0, 0), (0, _SC_D - EMB_DIM)))
    b2 = b.reshape(1, VOCAB)
    wt = W.T

    e = _make_sc_gather()(table_pad, x0)
    lse = _lse_pass(e, wt, b2)
    out_t = _out_pass(e, wt, b2, lse)
    return out_t.T.reshape(BATCH, 1, VOCAB)


# speedup vs baseline: 3.7634x; 1.0711x over previous
import functools

import jax
import jax.numpy as jnp
from jax import lax
from jax.experimental import pallas as pl
from jax.experimental.pallas import tpu as pltpu
from jax.experimental.pallas import tpu_sc as plsc

N_COLORS = 1000
EMB_DIM = 50
VOCAB = 100000
BATCH = 1024

V_TILE = 4096
NV = (VOCAB + V_TILE - 1) // V_TILE
B_HALF = BATCH // 2

NEG_BIG = -1e30


_SC_TILES = 32
_B_PER_TILE = BATCH // _SC_TILES
_SC_D = 128


@functools.cache
def _make_sc_gather():
    mesh = plsc.VectorSubcoreMesh(core_axis_name="c", subcore_axis_name="s")

    @functools.partial(
        pl.kernel,
        mesh=mesh,
        out_type=jax.ShapeDtypeStruct((BATCH, _SC_D), jnp.float32),
        scratch_types=[
            pltpu.VMEM((_B_PER_TILE,), jnp.int32),
            pltpu.VMEM((_B_PER_TILE, _SC_D), jnp.float32),
            pltpu.SemaphoreType.DMA,
        ],
    )
    def _sc_gather(table_hbm, idx_hbm, out_hbm, idx_v, rows_v, sem):
        wid = lax.axis_index("s") * 2 + lax.axis_index("c")
        base = wid * _B_PER_TILE
        pltpu.sync_copy(idx_hbm.at[pl.ds(base, _B_PER_TILE)], idx_v)
        pltpu.async_copy(table_hbm.at[idx_v], rows_v, sem).wait()
        pltpu.sync_copy(rows_v, out_hbm.at[pl.ds(base, _B_PER_TILE)])

    return _sc_gather


def _leaky_logits_t(e_ref, wt_ref, b_ref):
    e = e_ref[:, :EMB_DIM]
    e = jnp.where(e >= 0, e, 0.01 * e)
    lt = lax.dot_general(
        wt_ref[...], e, (((0,), (1,)), ((), ())),
        preferred_element_type=jnp.float32,
    )
    return lt + b_ref[...].reshape(V_TILE, 1)


def _lse_body(e_ref, w_ref, b_ref, lse_ref, m_ref, s_ref):
    j = pl.program_id(1)

    @pl.when(j == 0)
    def _():
        m_ref[...] = jnp.full_like(m_ref, -jnp.inf)
        s_ref[...] = jnp.zeros_like(s_ref)

    logits = _leaky_logits_t(e_ref, w_ref, b_ref)
    row = j * V_TILE + lax.broadcasted_iota(jnp.int32, logits.shape, 0)
    logits = jnp.where(row < VOCAB, logits, NEG_BIG)
    m_old = m_ref[...]
    m_new = jnp.maximum(m_old, jnp.max(logits, axis=0, keepdims=True))
    s_ref[...] = s_ref[...] * jnp.exp(m_old - m_new) + jnp.sum(
        jnp.exp(logits - m_new), axis=0, keepdims=True)
    m_ref[...] = m_new

    @pl.when(j == pl.num_programs(1) - 1)
    def _():
        lse_ref[...] = m_ref[...] + jnp.log(s_ref[...])


def _lse_pass(e, W, b2):
    return pl.pallas_call(
        _lse_body,
        grid=(2, NV),
        in_specs=[
            pl.BlockSpec((B_HALF, _SC_D), lambda i, j: (i, 0)),
            pl.BlockSpec((EMB_DIM, V_TILE), lambda i, j: (0, j)),
            pl.BlockSpec((1, V_TILE), lambda i, j: (0, j)),
        ],
        out_specs=pl.BlockSpec((1, B_HALF), lambda i, j: (0, i)),
        out_shape=jax.ShapeDtypeStruct((1, BATCH), jnp.float32),
        scratch_shapes=[
            pltpu.VMEM((1, B_HALF), jnp.float32),
            pltpu.VMEM((1, B_HALF), jnp.float32),
        ],
        compiler_params=pltpu.CompilerParams(
            dimension_semantics=("parallel", "arbitrary")),
    )(e, W, b2)


def _out_body(e_ref, w_ref, b_ref, lse_ref, o_ref):
    o_ref[...] = _leaky_logits_t(e_ref, w_ref, b_ref) - lse_ref[...]


def _out_pass(e, W, b2, lse):
    return pl.pallas_call(
        _out_body,
        grid=(2, NV),
        in_specs=[
            pl.BlockSpec((B_HALF, _SC_D), lambda i, j: (i, 0)),
            pl.BlockSpec((EMB_DIM, V_TILE), lambda i, j: (0, j)),
            pl.BlockSpec((1, V_TILE), lambda i, j: (0, j)),
            pl.BlockSpec((1, B_HALF), lambda i, j: (0, i)),
        ],
        out_specs=pl.BlockSpec((V_TILE, B_HALF), lambda i, j: (j, i)),
        out_shape=jax.ShapeDtypeStruct((VOCAB, BATCH), jnp.float32),
        compiler_params=pltpu.CompilerParams(
            dimension_semantics=("parallel", "arbitrary")),
    )(e, W, b2, lse)


def kernel(x, emb_table, W, b):
    x0 = x[:, 0].astype(jnp.int32)
    table_pad = jnp.pad(emb_table, ((0, 0), (0, _SC_D - EMB_DIM)))
    b2 = b.reshape(1, VOCAB)
    wt = W.T

    e = _make_sc_gather()(table_pad, x0)
    lse = _lse_pass(e, wt, b2)
    out_t = _out_pass(e, wt, b2, lse)
    return out_t.T.reshape(BATCH, 1, VOCAB)


# speedup vs baseline: 3.7889x; 1.0068x over previous
import functools

import jax
import jax.numpy as jnp
from jax import lax
from jax.experimental import pallas as pl
from jax.experimental.pallas import tpu as pltpu
from jax.experimental.pallas import tpu_sc as plsc

N_COLORS = 1000
EMB_DIM = 50
VOCAB = 100000
BATCH = 1024

V_TILE = 8192
NV = (VOCAB + V_TILE - 1) // V_TILE
B_HALF = BATCH // 2

NEG_BIG = -1e30


_SC_TILES = 32
_B_PER_TILE = BATCH // _SC_TILES
_SC_D = 128


@functools.cache
def _make_sc_gather():
    mesh = plsc.VectorSubcoreMesh(core_axis_name="c", subcore_axis_name="s")

    @functools.partial(
        pl.kernel,
        mesh=mesh,
        out_type=jax.ShapeDtypeStruct((BATCH, _SC_D), jnp.float32),
        scratch_types=[
            pltpu.VMEM((_B_PER_TILE,), jnp.int32),
            pltpu.VMEM((_B_PER_TILE, _SC_D), jnp.float32),
            pltpu.SemaphoreType.DMA,
        ],
    )
    def _sc_gather(table_hbm, idx_hbm, out_hbm, idx_v, rows_v, sem):
        wid = lax.axis_index("s") * 2 + lax.axis_index("c")
        base = wid * _B_PER_TILE
        pltpu.sync_copy(idx_hbm.at[pl.ds(base, _B_PER_TILE)], idx_v)
        pltpu.async_copy(table_hbm.at[idx_v], rows_v, sem).wait()
        pltpu.sync_copy(rows_v, out_hbm.at[pl.ds(base, _B_PER_TILE)])

    return _sc_gather


def _leaky_logits_t(e_ref, wt_ref, b_ref):
    e = e_ref[:, :EMB_DIM]
    e = jnp.where(e >= 0, e, 0.01 * e)
    lt = lax.dot_general(
        wt_ref[...], e, (((0,), (1,)), ((), ())),
        preferred_element_type=jnp.float32,
    )
    return lt + b_ref[...].reshape(V_TILE, 1)


def _lse_body(e_ref, w_ref, b_ref, lse_ref, m_ref, s_ref):
    j = pl.program_id(1)

    @pl.when(j == 0)
    def _():
        m_ref[...] = jnp.full_like(m_ref, -jnp.inf)
        s_ref[...] = jnp.zeros_like(s_ref)

    logits = _leaky_logits_t(e_ref, w_ref, b_ref)
    row = j * V_TILE + lax.broadcasted_iota(jnp.int32, logits.shape, 0)
    logits = jnp.where(row < VOCAB, logits, NEG_BIG)
    m_old = m_ref[...]
    m_new = jnp.maximum(m_old, jnp.max(logits, axis=0, keepdims=True))
    s_ref[...] = s_ref[...] * jnp.exp(m_old - m_new) + jnp.sum(
        jnp.exp(logits - m_new), axis=0, keepdims=True)
    m_ref[...] = m_new

    @pl.when(j == pl.num_programs(1) - 1)
    def _():
        lse_ref[...] = m_ref[...] + jnp.log(s_ref[...])


def _lse_pass(e, W, b2):
    return pl.pallas_call(
        _lse_body,
        grid=(2, NV),
        in_specs=[
            pl.BlockSpec((B_HALF, _SC_D), lambda i, j: (i, 0)),
            pl.BlockSpec((EMB_DIM, V_TILE), lambda i, j: (0, j)),
            pl.BlockSpec((1, V_TILE), lambda i, j: (0, j)),
        ],
        out_specs=pl.BlockSpec((1, B_HALF), lambda i, j: (0, i)),
        out_shape=jax.ShapeDtypeStruct((1, BATCH), jnp.float32),
        scratch_shapes=[
            pltpu.VMEM((1, B_HALF), jnp.float32),
            pltpu.VMEM((1, B_HALF), jnp.float32),
        ],
        compiler_params=pltpu.CompilerParams(
            dimension_semantics=("parallel", "arbitrary")),
    )(e, W, b2)


def _out_body(e_ref, w_ref, b_ref, lse_ref, o_ref):
    o_ref[...] = _leaky_logits_t(e_ref, w_ref, b_ref) - lse_ref[...]


def _out_pass(e, W, b2, lse):
    return pl.pallas_call(
        _out_body,
        grid=(2, NV),
        in_specs=[
            pl.BlockSpec((B_HALF, _SC_D), lambda i, j: (i, 0)),
            pl.BlockSpec((EMB_DIM, V_TILE), lambda i, j: (0, j)),
            pl.BlockSpec((1, V_TILE), lambda i, j: (0, j)),
            pl.BlockSpec((1, B_HALF), lambda i, j: (0, i)),
        ],
        out_specs=pl.BlockSpec((V_TILE, B_HALF), lambda i, j: (j, i)),
        out_shape=jax.ShapeDtypeStruct((VOCAB, BATCH), jnp.float32),
        compiler_params=pltpu.CompilerParams(
            dimension_semantics=("parallel", "arbitrary")),
    )(e, W, b2, lse)


def kernel(x, emb_table, W, b):
    x0 = x[:, 0].astype(jnp.int32)
    table_pad = jnp.pad(emb_table, ((0, 0), (0, _SC_D - EMB_DIM)))
    b2 = b.reshape(1, VOCAB)
    wt = W.T

    e = _make_sc_gather()(table_pad, x0)
    lse = _lse_pass(e, wt, b2)
    out_t = _out_pass(e, wt, b2, lse)
    return out_t.T.reshape(BATCH, 1, VOCAB)


# speedup vs baseline: 4.2281x; 1.1159x over previous
import functools

import jax
import jax.numpy as jnp
from jax import lax
from jax.experimental import pallas as pl
from jax.experimental.pallas import tpu as pltpu
from jax.experimental.pallas import tpu_sc as plsc

N_COLORS = 1000
EMB_DIM = 50
VOCAB = 100000
BATCH = 1024

V_TILE = 4096
NV = (VOCAB + V_TILE - 1) // V_TILE
B_HALF = BATCH // 1

NEG_BIG = -1e30


_SC_TILES = 32
_B_PER_TILE = BATCH // _SC_TILES
_SC_D = 128


@functools.cache
def _make_sc_gather():
    mesh = plsc.VectorSubcoreMesh(core_axis_name="c", subcore_axis_name="s")

    @functools.partial(
        pl.kernel,
        mesh=mesh,
        out_type=jax.ShapeDtypeStruct((BATCH, _SC_D), jnp.float32),
        scratch_types=[
            pltpu.VMEM((_B_PER_TILE,), jnp.int32),
            pltpu.VMEM((_B_PER_TILE, _SC_D), jnp.float32),
            pltpu.SemaphoreType.DMA,
        ],
    )
    def _sc_gather(table_hbm, idx_hbm, out_hbm, idx_v, rows_v, sem):
        wid = lax.axis_index("s") * 2 + lax.axis_index("c")
        base = wid * _B_PER_TILE
        pltpu.sync_copy(idx_hbm.at[pl.ds(base, _B_PER_TILE)], idx_v)
        pltpu.async_copy(table_hbm.at[idx_v], rows_v, sem).wait()
        pltpu.sync_copy(rows_v, out_hbm.at[pl.ds(base, _B_PER_TILE)])

    return _sc_gather


def _leaky_logits_t(e_ref, wt_ref, b_ref):
    e = e_ref[:, :EMB_DIM]
    e = jnp.where(e >= 0, e, 0.01 * e)
    lt = lax.dot_general(
        wt_ref[...], e, (((0,), (1,)), ((), ())),
        preferred_element_type=jnp.float32,
    )
    return lt + b_ref[...].reshape(V_TILE, 1)


def _lse_body(e_ref, w_ref, b_ref, lse_ref, m_ref, s_ref):
    j = pl.program_id(1)

    @pl.when(j == 0)
    def _():
        m_ref[...] = jnp.full_like(m_ref, -jnp.inf)
        s_ref[...] = jnp.zeros_like(s_ref)

    logits = _leaky_logits_t(e_ref, w_ref, b_ref)
    row = j * V_TILE + lax.broadcasted_iota(jnp.int32, logits.shape, 0)
    logits = jnp.where(row < VOCAB, logits, NEG_BIG)
    m_old = m_ref[...]
    m_new = jnp.maximum(m_old, jnp.max(logits, axis=0, keepdims=True))
    s_ref[...] = s_ref[...] * jnp.exp(m_old - m_new) + jnp.sum(
        jnp.exp(logits - m_new), axis=0, keepdims=True)
    m_ref[...] = m_new

    @pl.when(j == pl.num_programs(1) - 1)
    def _():
        lse_ref[...] = m_ref[...] + jnp.log(s_ref[...])


def _lse_pass(e, W, b2):
    return pl.pallas_call(
        _lse_body,
        grid=(1, NV),
        in_specs=[
            pl.BlockSpec((B_HALF, _SC_D), lambda i, j: (i, 0)),
            pl.BlockSpec((EMB_DIM, V_TILE), lambda i, j: (0, j)),
            pl.BlockSpec((1, V_TILE), lambda i, j: (0, j)),
        ],
        out_specs=pl.BlockSpec((1, B_HALF), lambda i, j: (0, i)),
        out_shape=jax.ShapeDtypeStruct((1, BATCH), jnp.float32),
        scratch_shapes=[
            pltpu.VMEM((1, B_HALF), jnp.float32),
            pltpu.VMEM((1, B_HALF), jnp.float32),
        ],
        compiler_params=pltpu.CompilerParams(
            dimension_semantics=("parallel", "arbitrary")),
    )(e, W, b2)


def _out_body(e_ref, w_ref, b_ref, lse_ref, o_ref):
    o_ref[...] = _leaky_logits_t(e_ref, w_ref, b_ref) - lse_ref[...]


def _out_pass(e, W, b2, lse):
    return pl.pallas_call(
        _out_body,
        grid=(1, NV),
        in_specs=[
            pl.BlockSpec((B_HALF, _SC_D), lambda i, j: (i, 0)),
            pl.BlockSpec((EMB_DIM, V_TILE), lambda i, j: (0, j)),
            pl.BlockSpec((1, V_TILE), lambda i, j: (0, j)),
            pl.BlockSpec((1, B_HALF), lambda i, j: (0, i)),
        ],
        out_specs=pl.BlockSpec((V_TILE, B_HALF), lambda i, j: (j, i)),
        out_shape=jax.ShapeDtypeStruct((VOCAB, BATCH), jnp.float32),
        compiler_params=pltpu.CompilerParams(
            dimension_semantics=("parallel", "arbitrary")),
    )(e, W, b2, lse)


def kernel(x, emb_table, W, b):
    x0 = x[:, 0].astype(jnp.int32)
    table_pad = jnp.pad(emb_table, ((0, 0), (0, _SC_D - EMB_DIM)))
    b2 = b.reshape(1, VOCAB)
    wt = W.T

    e = _make_sc_gather()(table_pad, x0)
    lse = _lse_pass(e, wt, b2)
    out_t = _out_pass(e, wt, b2, lse)
    return out_t.T.reshape(BATCH, 1, VOCAB)
